# baseline (speedup 1.0000x reference)
import jax
import jax.numpy as jnp
from jax import lax
from jax.experimental import pallas as pl
from jax.experimental.pallas import tpu as pltpu

_R = 70
_H = 512
_S = 128
_STRIDE = 3
_CHUNK = 80
_CW = 256
_GRID = 2
_SLOTS = 6
_DEPTH = 5


def _propose_kernel(score_ref, fake_hbm, reala_hbm,
                    fbr_ref, rar_ref, fbf_ref, raf_ref, fabf_ref, rabr_ref,
                    buf_ref, sem):
    g = pl.program_id(0)
    per = fbr_ref.shape[0]

    s = score_ref[:, 0]
    ri = lax.broadcasted_iota(jnp.int32, (_S, _S), 0)
    ci = lax.broadcasted_iota(jnp.int32, (_S, _S), 1)
    flat = (ri * _S + ci)[None]
    big = jnp.int32(1 << 30)
    vmax = jnp.max(s, axis=(1, 2))
    vmin = jnp.min(s, axis=(1, 2))
    imax = jnp.min(jnp.where(s == vmax[:, None, None], flat, big), axis=(1, 2))
    imin = jnp.min(jnp.where(s == vmin[:, None, None], flat, big), axis=(1, 2))
    rr = jnp.where(vmax > 0.0, imax // _S, 0) * _STRIDE + _R
    cr = jnp.where(vmax > 0.0, imax % _S, 0) * _STRIDE + _R
    rf = jnp.where(vmin < 1.0, imin // _S, 1) * _STRIDE + _R
    cf = jnp.where(vmin < 1.0, imin % _S, 1) * _STRIDE + _R
    cm = jnp.stack([rr, cr, rf, cf], axis=0)
    coord = [[cm[q, j] for q in range(4)] for j in range(per)]

    def bases(r0, c0):
        ra = jnp.minimum((r0 >> 3) << 3, _H - _CHUNK)
        ca = jnp.minimum((c0 >> 7) << 7, _H - _CW)
        return pl.multiple_of(ra, 8), pl.multiple_of(ca, 128)

    def issue(j, slot):
        bb = g * per + j
        rr_, cr_, rf_, cf_ = coord[j]
        for cs, (r0, c0) in enumerate(((rr_, cr_), (rf_, cf_))):
            ra, ca = bases(r0, c0)
            for im, img in enumerate((fake_hbm, reala_hbm)):
                for c in range(3):
                    pltpu.make_async_copy(
                        img.at[bb, c, pl.ds(ra, _CHUNK), pl.ds(ca, _CW)],
                        buf_ref.at[slot, cs,
                                   pl.ds(im * 3 * _CHUNK + c * _CHUNK, _CHUNK)],
                        sem.at[slot]).start()

    for k in range(min(_DEPTH, per)):
        issue(k, k % _SLOTS)
    for j in range(per):
        slot = j % _SLOTS
        if j + _DEPTH < per:
            issue(j + _DEPTH, (j + _DEPTH) % _SLOTS)
        pltpu.make_async_copy(buf_ref.at[slot], buf_ref.at[slot],
                              sem.at[slot]).wait()
        rr_, cr_, rf_, cf_ = coord[j]
        crops = []
        for cs, (r0, c0) in enumerate(((rr_, cr_), (rf_, cf_))):
            ra, ca = bases(r0, c0)
            jc = lax.broadcasted_iota(jnp.int32, (_CW, _R), 0)
            kc = lax.broadcasted_iota(jnp.int32, (_CW, _R), 1)
            csel = (ca + jc == c0 + kc).astype(jnp.float32)
            ir = lax.broadcasted_iota(jnp.int32, (_R, _CHUNK), 0)
            jr = lax.broadcasted_iota(jnp.int32, (_R, _CHUNK), 1)
            rsel = ((ra + jr == r0 + ir)
                    & (r0 + ir <= _H - 1)).astype(jnp.float32)
            t = jnp.dot(buf_ref[slot, cs], csel,
                        preferred_element_type=jnp.float32)
            crops.append([
                jnp.dot(rsel, t[k * _CHUNK:(k + 1) * _CHUNK],
                        preferred_element_type=jnp.float32)
                for k in range(6)])
        fbr, rar = crops[0][:3], crops[0][3:]
        fbf, raf = crops[1][:3], crops[1][3:]
        for c in range(3):
            fbr_ref[j, c] = fbr[c]
            rar_ref[j, c] = rar[c]
            fbf_ref[j, c] = fbf[c]
            raf_ref[j, c] = raf[c]
            fabf_ref[j, c] = raf[c]
            fabf_ref[j, 3 + c] = fbf[c]
            rabr_ref[j, c] = rar[c]
            rabr_ref[j, 3 + c] = fbr[c]


def kernel(real_B, fake_B, real_A, score_map):
    del real_B
    B = score_map.shape[0]
    per = B // _GRID
    f32 = jnp.float32
    crop3 = jax.ShapeDtypeStruct((B, 3, _R, _R), f32)
    crop6 = jax.ShapeDtypeStruct((B, 6, _R, _R), f32)
    spec3 = pl.BlockSpec((per, 3, _R, _R), lambda g: (g, 0, 0, 0))
    spec6 = pl.BlockSpec((per, 6, _R, _R), lambda g: (g, 0, 0, 0))
    outs = pl.pallas_call(
        _propose_kernel,
        out_shape=(crop3, crop3, crop3, crop3, crop6, crop6),
        grid=(_GRID,),
        in_specs=[
            pl.BlockSpec((per, 1, _S, _S), lambda g: (g, 0, 0, 0)),
            pl.BlockSpec(memory_space=pl.ANY),
            pl.BlockSpec(memory_space=pl.ANY),
        ],
        out_specs=(spec3, spec3, spec3, spec3, spec6, spec6),
        scratch_shapes=[
            pltpu.VMEM((_SLOTS, 2, 2 * 3 * _CHUNK, _CW), f32),
            pltpu.SemaphoreType.DMA((_SLOTS,)),
        ],
        compiler_params=pltpu.CompilerParams(
            dimension_semantics=("parallel",),
        ),
        name="drpan_proposal",
    )(score_map, fake_B, real_A)
    return tuple(outs)
